# Initial kernel scaffold; baseline (speedup 1.0000x reference)
import jax
import jax.numpy as jnp
from jax.experimental import pallas as pl


def kernel(input, adj, W_gcn, b_gcn, ln1_g, ln1_b, W_ih, W_hh, b_ih, b_hh, ln2_g, ln2_b, W_skip, b_skip):
    raise NotImplementedError("write your pallas kernel here")


# speedup vs baseline: 1.0507x; 1.0507x over previous
import functools

import jax
import jax.numpy as jnp
from jax.experimental import pallas as pl
from jax.experimental.pallas import tpu as pltpu

EPS = 1e-5


def _support_kernel(inp_ref, w_ref, out_ref):
    out_ref[...] = jnp.dot(inp_ref[...], w_ref[...],
                           preferred_element_type=jnp.float32)


def _main_kernel(adj_ref, sup_ref, bgcn_ref, ln1g_ref, ln1b_ref,
                 wih_t_ref, brnn_ref, ln2g_ref, ln2b_ref,
                 wsh_t_ref, wsx_t_ref, bskip_ref, out_ref):
    x = jnp.dot(adj_ref[...], sup_ref[...],
                preferred_element_type=jnp.float32) + bgcn_ref[...]
    x = jnp.maximum(x, 0.0)
    mu = jnp.mean(x, axis=-1, keepdims=True)
    var = jnp.mean((x - mu) ** 2, axis=-1, keepdims=True)
    x = (x - mu) * jax.lax.rsqrt(var + EPS) * ln1g_ref[...] + ln1b_ref[...]
    h = jnp.tanh(jnp.dot(x, wih_t_ref[...],
                         preferred_element_type=jnp.float32)
                 + brnn_ref[...])
    mu2 = jnp.mean(h, axis=-1, keepdims=True)
    var2 = jnp.mean((h - mu2) ** 2, axis=-1, keepdims=True)
    h = (h - mu2) * jax.lax.rsqrt(var2 + EPS) * ln2g_ref[...] + ln2b_ref[...]
    y = (jnp.dot(h, wsh_t_ref[...], preferred_element_type=jnp.float32)
         + jnp.dot(x, wsx_t_ref[...], preferred_element_type=jnp.float32)
         + bskip_ref[...])
    out_ref[...] = jnp.where(y >= 0.0, y, 0.01 * y)


def kernel(input, adj, W_gcn, b_gcn, ln1_g, ln1_b, W_ih, W_hh, b_ih, b_hh,
           ln2_g, ln2_b, W_skip, b_skip):
    n, din = input.shape
    dout = W_gcn.shape[1]

    bs = 1000
    support = pl.pallas_call(
        _support_kernel,
        grid=(n // bs,),
        in_specs=[
            pl.BlockSpec((bs, din), lambda i: (i, 0)),
            pl.BlockSpec((din, dout), lambda i: (0, 0)),
        ],
        out_specs=pl.BlockSpec((bs, dout), lambda i: (i, 0)),
        out_shape=jax.ShapeDtypeStruct((n, dout), jnp.float32),
    )(input, W_gcn)

    bgcn = b_gcn.reshape(1, dout)
    ln1g = ln1_g.reshape(1, dout)
    ln1b = ln1_b.reshape(1, dout)
    brnn = (b_ih + b_hh).reshape(1, dout)
    ln2g = ln2_g.reshape(1, dout)
    ln2b = ln2_b.reshape(1, dout)
    wih_t = W_ih.T
    wsh_t = W_skip[:, :dout].T
    wsx_t = W_skip[:, dout:].T
    bskip = b_skip.reshape(1, dout)

    bm = 200
    nm = n // bm
    full = lambda i: (0, 0)

    out = pl.pallas_call(
        _main_kernel,
        grid=(nm,),
        in_specs=[
            pl.BlockSpec((bm, n), lambda i: (i, 0)),
            pl.BlockSpec((n, dout), full),
            pl.BlockSpec((1, dout), full),
            pl.BlockSpec((1, dout), full),
            pl.BlockSpec((1, dout), full),
            pl.BlockSpec((dout, dout), full),
            pl.BlockSpec((1, dout), full),
            pl.BlockSpec((1, dout), full),
            pl.BlockSpec((1, dout), full),
            pl.BlockSpec((dout, dout), full),
            pl.BlockSpec((dout, dout), full),
            pl.BlockSpec((1, dout), full),
        ],
        out_specs=pl.BlockSpec((bm, dout), lambda i: (i, 0)),
        out_shape=jax.ShapeDtypeStruct((n, dout), jnp.float32),
        compiler_params=pltpu.CompilerParams(
            dimension_semantics=("arbitrary",),
        ),
    )(adj, support, bgcn, ln1g, ln1b, wih_t, brnn, ln2g, ln2b,
      wsh_t, wsx_t, bskip)
    return out


# speedup vs baseline: 1.1526x; 1.0970x over previous
import functools

import jax
import jax.numpy as jnp
from jax.experimental import pallas as pl
from jax.experimental.pallas import tpu as pltpu

EPS = 1e-5


def _support_kernel(inp_ref, w_ref, out_ref):
    out_ref[...] = jnp.dot(inp_ref[...], w_ref[...],
                           preferred_element_type=jnp.float32)


def _main_kernel(adj_ref, sup_ref, bgcn_ref, ln1g_ref, ln1b_ref,
                 wih_t_ref, brnn_ref, ln2g_ref, ln2b_ref,
                 wsh_t_ref, wsx_t_ref, bskip_ref, out_ref):
    x = jnp.dot(adj_ref[...], sup_ref[...],
                preferred_element_type=jnp.float32) + bgcn_ref[...]
    x = jnp.maximum(x, 0.0)
    mu = jnp.mean(x, axis=-1, keepdims=True)
    var = jnp.mean((x - mu) ** 2, axis=-1, keepdims=True)
    x = (x - mu) * jax.lax.rsqrt(var + EPS) * ln1g_ref[...] + ln1b_ref[...]
    h = jnp.tanh(jnp.dot(x, wih_t_ref[...],
                         preferred_element_type=jnp.float32)
                 + brnn_ref[...])
    mu2 = jnp.mean(h, axis=-1, keepdims=True)
    var2 = jnp.mean((h - mu2) ** 2, axis=-1, keepdims=True)
    h = (h - mu2) * jax.lax.rsqrt(var2 + EPS) * ln2g_ref[...] + ln2b_ref[...]
    y = (jnp.dot(h, wsh_t_ref[...], preferred_element_type=jnp.float32)
         + jnp.dot(x, wsx_t_ref[...], preferred_element_type=jnp.float32)
         + bskip_ref[...])
    out_ref[...] = jnp.where(y >= 0.0, y, 0.01 * y)


def kernel(input, adj, W_gcn, b_gcn, ln1_g, ln1_b, W_ih, W_hh, b_ih, b_hh,
           ln2_g, ln2_b, W_skip, b_skip):
    n, din = input.shape
    dout = W_gcn.shape[1]

    bs = 1000
    support = pl.pallas_call(
        _support_kernel,
        grid=(n // bs,),
        in_specs=[
            pl.BlockSpec((bs, din), lambda i: (i, 0)),
            pl.BlockSpec((din, dout), lambda i: (0, 0)),
        ],
        out_specs=pl.BlockSpec((bs, dout), lambda i: (i, 0)),
        out_shape=jax.ShapeDtypeStruct((n, dout), jnp.float32),
    )(input, W_gcn)

    bgcn = b_gcn.reshape(1, dout)
    ln1g = ln1_g.reshape(1, dout)
    ln1b = ln1_b.reshape(1, dout)
    brnn = (b_ih + b_hh).reshape(1, dout)
    ln2g = ln2_g.reshape(1, dout)
    ln2b = ln2_b.reshape(1, dout)
    wih_t = W_ih.T
    wsh_t = W_skip[:, :dout].T
    wsx_t = W_skip[:, dout:].T
    bskip = b_skip.reshape(1, dout)

    bm = 400
    nm = n // bm
    full = lambda i: (0, 0)

    out = pl.pallas_call(
        _main_kernel,
        grid=(nm,),
        in_specs=[
            pl.BlockSpec((bm, n), lambda i: (i, 0)),
            pl.BlockSpec((n, dout), full),
            pl.BlockSpec((1, dout), full),
            pl.BlockSpec((1, dout), full),
            pl.BlockSpec((1, dout), full),
            pl.BlockSpec((dout, dout), full),
            pl.BlockSpec((1, dout), full),
            pl.BlockSpec((1, dout), full),
            pl.BlockSpec((1, dout), full),
            pl.BlockSpec((dout, dout), full),
            pl.BlockSpec((dout, dout), full),
            pl.BlockSpec((1, dout), full),
        ],
        out_specs=pl.BlockSpec((bm, dout), lambda i: (i, 0)),
        out_shape=jax.ShapeDtypeStruct((n, dout), jnp.float32),
        compiler_params=pltpu.CompilerParams(
            dimension_semantics=("parallel",),
        ),
    )(adj, support, bgcn, ln1g, ln1b, wih_t, brnn, ln2g, ln2b,
      wsh_t, wsx_t, bskip)
    return out


# speedup vs baseline: 1.1529x; 1.0002x over previous
import functools

import jax
import jax.numpy as jnp
from jax.experimental import pallas as pl
from jax.experimental.pallas import tpu as pltpu

EPS = 1e-5


def _support_kernel(inp_ref, w_ref, out_ref):
    out_ref[...] = jnp.dot(inp_ref[...], w_ref[...],
                           preferred_element_type=jnp.float32)


def _main_kernel(adj_ref, sup_ref, bgcn_ref, ln1g_ref, ln1b_ref,
                 wih_t_ref, brnn_ref, ln2g_ref, ln2b_ref,
                 wsh_t_ref, wsx_t_ref, bskip_ref, out_ref):
    x = jnp.dot(adj_ref[...], sup_ref[...],
                preferred_element_type=jnp.float32) + bgcn_ref[...]
    x = jnp.maximum(x, 0.0)
    mu = jnp.mean(x, axis=-1, keepdims=True)
    var = jnp.mean((x - mu) ** 2, axis=-1, keepdims=True)
    x = (x - mu) * jax.lax.rsqrt(var + EPS) * ln1g_ref[...] + ln1b_ref[...]
    h = jnp.tanh(jnp.dot(x, wih_t_ref[...],
                         preferred_element_type=jnp.float32)
                 + brnn_ref[...])
    mu2 = jnp.mean(h, axis=-1, keepdims=True)
    var2 = jnp.mean((h - mu2) ** 2, axis=-1, keepdims=True)
    h = (h - mu2) * jax.lax.rsqrt(var2 + EPS) * ln2g_ref[...] + ln2b_ref[...]
    y = (jnp.dot(h, wsh_t_ref[...], preferred_element_type=jnp.float32)
         + jnp.dot(x, wsx_t_ref[...], preferred_element_type=jnp.float32)
         + bskip_ref[...])
    out_ref[...] = jnp.where(y >= 0.0, y, 0.01 * y)


def kernel(input, adj, W_gcn, b_gcn, ln1_g, ln1_b, W_ih, W_hh, b_ih, b_hh,
           ln2_g, ln2_b, W_skip, b_skip):
    n, din = input.shape
    dout = W_gcn.shape[1]

    bs = 1000
    support = pl.pallas_call(
        _support_kernel,
        grid=(n // bs,),
        in_specs=[
            pl.BlockSpec((bs, din), lambda i: (i, 0)),
            pl.BlockSpec((din, dout), lambda i: (0, 0)),
        ],
        out_specs=pl.BlockSpec((bs, dout), lambda i: (i, 0)),
        out_shape=jax.ShapeDtypeStruct((n, dout), jnp.float32),
    )(input, W_gcn)

    bgcn = b_gcn.reshape(1, dout)
    ln1g = ln1_g.reshape(1, dout)
    ln1b = ln1_b.reshape(1, dout)
    brnn = (b_ih + b_hh).reshape(1, dout)
    ln2g = ln2_g.reshape(1, dout)
    ln2b = ln2_b.reshape(1, dout)
    wih_t = W_ih.T
    wsh_t = W_skip[:, :dout].T
    wsx_t = W_skip[:, dout:].T
    bskip = b_skip.reshape(1, dout)

    bm = 400
    nm = n // bm
    full = lambda i: (0, 0)

    out = pl.pallas_call(
        _main_kernel,
        grid=(nm,),
        in_specs=[
            pl.BlockSpec((bm, n), lambda i: (i, 0)),
            pl.BlockSpec((n, dout), full),
            pl.BlockSpec((1, dout), full),
            pl.BlockSpec((1, dout), full),
            pl.BlockSpec((1, dout), full),
            pl.BlockSpec((dout, dout), full),
            pl.BlockSpec((1, dout), full),
            pl.BlockSpec((1, dout), full),
            pl.BlockSpec((1, dout), full),
            pl.BlockSpec((dout, dout), full),
            pl.BlockSpec((dout, dout), full),
            pl.BlockSpec((1, dout), full),
        ],
        out_specs=pl.BlockSpec((bm, dout), lambda i: (i, 0)),
        out_shape=jax.ShapeDtypeStruct((n, dout), jnp.float32),
        compiler_params=pltpu.CompilerParams(
            dimension_semantics=("arbitrary",),
        ),
    )(adj, support, bgcn, ln1g, ln1b, wih_t, brnn, ln2g, ln2b,
      wsh_t, wsx_t, bskip)
    return out


# speedup vs baseline: 1.2322x; 1.0688x over previous
import jax
import jax.numpy as jnp
from jax.experimental import pallas as pl
from jax.experimental.pallas import tpu as pltpu

EPS = 1e-5


def _main_kernel(inp_ref, wgcn_ref, adj_ref, bgcn_ref, ln1g_ref, ln1b_ref,
                 wih_t_ref, brnn_ref, ln2g_ref, ln2b_ref,
                 wsh_t_ref, wsx_t_ref, bskip_ref, out_ref, sup_ref):
    @pl.when(pl.program_id(0) == 0)
    def _compute_support():
        sup_ref[...] = jnp.dot(inp_ref[...], wgcn_ref[...],
                               preferred_element_type=jnp.float32)

    x = jnp.dot(adj_ref[...], sup_ref[...],
                preferred_element_type=jnp.float32) + bgcn_ref[...]
    x = jnp.maximum(x, 0.0)
    mu = jnp.mean(x, axis=-1, keepdims=True)
    var = jnp.mean((x - mu) ** 2, axis=-1, keepdims=True)
    x = (x - mu) * jax.lax.rsqrt(var + EPS) * ln1g_ref[...] + ln1b_ref[...]
    h = jnp.tanh(jnp.dot(x, wih_t_ref[...],
                         preferred_element_type=jnp.float32)
                 + brnn_ref[...])
    mu2 = jnp.mean(h, axis=-1, keepdims=True)
    var2 = jnp.mean((h - mu2) ** 2, axis=-1, keepdims=True)
    h = (h - mu2) * jax.lax.rsqrt(var2 + EPS) * ln2g_ref[...] + ln2b_ref[...]
    y = (jnp.dot(h, wsh_t_ref[...], preferred_element_type=jnp.float32)
         + jnp.dot(x, wsx_t_ref[...], preferred_element_type=jnp.float32)
         + bskip_ref[...])
    out_ref[...] = jnp.where(y >= 0.0, y, 0.01 * y)


def kernel(input, adj, W_gcn, b_gcn, ln1_g, ln1_b, W_ih, W_hh, b_ih, b_hh,
           ln2_g, ln2_b, W_skip, b_skip):
    n, din = input.shape
    dout = W_gcn.shape[1]

    bgcn = b_gcn.reshape(1, dout)
    ln1g = ln1_g.reshape(1, dout)
    ln1b = ln1_b.reshape(1, dout)
    brnn = (b_ih + b_hh).reshape(1, dout)
    ln2g = ln2_g.reshape(1, dout)
    ln2b = ln2_b.reshape(1, dout)
    wih_t = W_ih.T
    wsh_t = W_skip[:, :dout].T
    wsx_t = W_skip[:, dout:].T
    bskip = b_skip.reshape(1, dout)

    bm = 400
    nm = n // bm
    full = lambda i: (0, 0)

    out = pl.pallas_call(
        _main_kernel,
        grid=(nm,),
        in_specs=[
            pl.BlockSpec((n, din), full),
            pl.BlockSpec((din, dout), full),
            pl.BlockSpec((bm, n), lambda i: (i, 0)),
            pl.BlockSpec((1, dout), full),
            pl.BlockSpec((1, dout), full),
            pl.BlockSpec((1, dout), full),
            pl.BlockSpec((dout, dout), full),
            pl.BlockSpec((1, dout), full),
            pl.BlockSpec((1, dout), full),
            pl.BlockSpec((1, dout), full),
            pl.BlockSpec((dout, dout), full),
            pl.BlockSpec((dout, dout), full),
            pl.BlockSpec((1, dout), full),
        ],
        out_specs=pl.BlockSpec((bm, dout), lambda i: (i, 0)),
        out_shape=jax.ShapeDtypeStruct((n, dout), jnp.float32),
        scratch_shapes=[pltpu.VMEM((n, dout), jnp.float32)],
        compiler_params=pltpu.CompilerParams(
            dimension_semantics=("arbitrary",),
        ),
    )(input, W_gcn, adj, bgcn, ln1g, ln1b, wih_t, brnn, ln2g, ln2b,
      wsh_t, wsx_t, bskip)
    return out
